# Initial kernel scaffold; baseline (speedup 1.0000x reference)
import jax
import jax.numpy as jnp
from jax.experimental import pallas as pl


def kernel(f, x, permutation_weights, gate_weights, bias):
    raise NotImplementedError("write your pallas kernel here")


# speedup vs baseline: 3.8906x; 3.8906x over previous
import functools

import jax
import jax.numpy as jnp
import numpy as np
from jax.experimental import pallas as pl
from jax.experimental.pallas import tpu as pltpu

E = 16
K = 2
N = 4096
D = 768
P = 4

BN = 256
NB = N // BN


def _gate_block(logits):
    bn = logits.shape[0]
    iota = jax.lax.broadcasted_iota(jnp.int32, (bn, E), 1)
    neg = jnp.float32(-np.inf)
    m1 = jnp.max(logits, axis=1, keepdims=True)
    i1 = jnp.min(jnp.where(logits == m1, iota, E), axis=1, keepdims=True)
    sel1 = iota == i1
    logits2 = jnp.where(sel1, neg, logits)
    m2 = jnp.max(logits2, axis=1, keepdims=True)
    i2 = jnp.min(jnp.where(logits2 == m2, iota, E), axis=1, keepdims=True)
    sel2 = iota == i2
    mv1 = jnp.where(m1 == 0.0, neg, m1)
    mv2 = jnp.where(m2 == 0.0, neg, m2)
    mm = jnp.maximum(mv1, mv2)
    a1 = jnp.exp(mv1 - mm)
    a2 = jnp.exp(mv2 - mm)
    denom = a1 + a2
    return (jnp.where(sel1, a1, 0.0) + jnp.where(sel2, a2, 0.0)) / denom


def _moe_kernel(x_ref, gw_ref, pw_ref, bias_ref, f_ref,
                y_ref, soft_ref, hard_ref, soft_acc, hard_acc):
    i = pl.program_id(0)

    logits = jnp.dot(x_ref[...], gw_ref[...].T,
                     preferred_element_type=jnp.float32) + bias_ref[...]
    g = _gate_block(logits)
    pw = jnp.mean(pw_ref[...], axis=0)
    w = jnp.dot(g, pw, preferred_element_type=jnp.float32)
    w = w / jnp.sum(w, axis=1, keepdims=True)

    acc = w[:, 0:1] * f_ref[0]
    for e in range(1, E):
        acc = acc + w[:, e:e + 1] * f_ref[e]
    y_ref[...] = acc

    @pl.when(i == 0)
    def _():
        soft_acc[...] = jnp.zeros_like(soft_acc)
        hard_acc[...] = jnp.zeros_like(hard_acc)

    soft_acc[...] += jnp.sum(w, axis=0, keepdims=True)
    hard_acc[...] += jnp.sum((w >= 1e-5).astype(jnp.float32), axis=0,
                             keepdims=True)

    @pl.when(i == NB - 1)
    def _():
        soft_ref[...] = soft_acc[...] / N
        hard_ref[...] = hard_acc[...] / N


@functools.partial(jax.jit, static_argnames=())
def kernel(f, x, permutation_weights, gate_weights, bias):
    bias2 = bias.reshape(1, E)
    y, soft, hard = pl.pallas_call(
        _moe_kernel,
        grid=(NB,),
        in_specs=[
            pl.BlockSpec((BN, D), lambda i: (i, 0)),
            pl.BlockSpec((E, D), lambda i: (0, 0)),
            pl.BlockSpec((P, E, E), lambda i: (0, 0, 0)),
            pl.BlockSpec((1, E), lambda i: (0, 0)),
            pl.BlockSpec((E, BN, D), lambda i: (0, i, 0)),
        ],
        out_specs=[
            pl.BlockSpec((BN, D), lambda i: (i, 0)),
            pl.BlockSpec((1, E), lambda i: (0, 0)),
            pl.BlockSpec((1, E), lambda i: (0, 0)),
        ],
        out_shape=[
            jax.ShapeDtypeStruct((N, D), jnp.float32),
            jax.ShapeDtypeStruct((1, E), jnp.float32),
            jax.ShapeDtypeStruct((1, E), jnp.float32),
        ],
        scratch_shapes=[
            pltpu.VMEM((1, E), jnp.float32),
            pltpu.VMEM((1, E), jnp.float32),
        ],
    )(x, gate_weights, permutation_weights, bias2, f)
    return y, soft.reshape(E, 1), hard.reshape(E, 1)
